# baseline (speedup 1.0000x reference)
import functools

import jax
import jax.numpy as jnp
from jax import lax
from jax.experimental import pallas as pl
from jax.experimental.pallas import tpu as pltpu
from jax.experimental.pallas import tpu_sc as plsc

_B, _S, _D = 4, 4096, 768
_N = _B * _S
_NW = 32
_P = _S // _NW
_C = 16
_SLABS = (8, 16, 16, 16, 16, 16, 16, 16, 8)
_OFFS = tuple(sum(_SLABS[:i]) for i in range(len(_SLABS)))
_NSLAB = len(_SLABS)
_NVREG = _D // 16


def _emb_kernel(x_flat, tok_table, pe):
    mesh = plsc.VectorSubcoreMesh(core_axis_name="c", subcore_axis_name="s")

    @functools.partial(
        pl.kernel,
        out_type=jax.ShapeDtypeStruct((_N, _D), jnp.float32),
        mesh=mesh,
        scratch_types=[
            [[pltpu.VMEM((_C, _D), jnp.float32) for _ in range(_B)]
             for _ in range(2)],
            [pltpu.VMEM((_C, _D), jnp.float32) for _ in range(2)],
            pltpu.VMEM((_B * _P,), jnp.int32),
            [pltpu.SemaphoreType.DMA for _ in range(2)],
            [pltpu.SemaphoreType.DMA for _ in range(2)],
            [pltpu.SemaphoreType.DMA for _ in range(2)],
            pltpu.SemaphoreType.DMA,
        ],
    )
    def body(x_hbm, table_hbm, pe_hbm, out_hbm, sets, pe_v, idx_v,
             gsems, wsems, pe_sems, idx_sem):
        wid = lax.axis_index("s") * 2 + lax.axis_index("c")
        s0 = wid * _P

        def start_gathers(t):
            p0, sz = _OFFS[t], _SLABS[t]
            return [pltpu.async_copy(
                        table_hbm.at[idx_v.at[pl.ds(b * _P + p0, sz)]],
                        sets[t % 2][b].at[pl.ds(0, sz)], gsems[t % 2])
                    for b in range(_B)]

        def start_writes(t):
            p0, sz = _OFFS[t], _SLABS[t]
            return [pltpu.async_copy(
                        sets[t % 2][b].at[pl.ds(0, sz)],
                        out_hbm.at[pl.ds(b * _S + s0 + p0, sz)],
                        wsems[t % 2])
                    for b in range(_B)]

        def start_pe_load(t):
            p0, sz = _OFFS[t], _SLABS[t]
            return pltpu.async_copy(
                pe_hbm.at[pl.ds(s0 + p0, sz)],
                pe_v[t % 2].at[pl.ds(0, sz)], pe_sems[t % 2])

        idx_d = [pltpu.async_copy(x_hbm.at[pl.ds(b * _S + s0, _P)],
                                  idx_v.at[pl.ds(b * _P, _P)], idx_sem)
                 for b in range(_B)]
        pe_d = {0: start_pe_load(0), 1: start_pe_load(1)}
        for d in idx_d:
            d.wait()
        gd = {0: start_gathers(0)}
        wd = {}

        for t in range(_NSLAB):
            for d in gd.pop(t):
                d.wait()
            if t + 1 < _NSLAB:
                if t >= 1:
                    for d in wd.pop(t - 1):
                        d.wait()
                gd[t + 1] = start_gathers(t + 1)
            pe_d.pop(t).wait()
            pev = pe_v[t % 2]
            bufs = sets[t % 2]

            @pl.loop(0, _SLABS[t])
            def _(r):
                for j in range(_NVREG):
                    v = pev[r, pl.ds(j * 16, 16)]
                    for b in range(_B):
                        plsc.addupdate(bufs[b].at[r, pl.ds(j * 16, 16)], v)

            if t + 2 < _NSLAB:
                pe_d[t + 2] = start_pe_load(t + 2)
            wd[t] = start_writes(t)
        for ds_ in wd.values():
            for d in ds_:
                d.wait()

    return body(x_flat, tok_table, pe)


def kernel(x, tok_table, pe):
    out = _emb_kernel(x.reshape(_N), tok_table, pe)
    return out.reshape(_B, _S, _D)


# speedup vs baseline: 1.0147x; 1.0147x over previous
import functools

import jax
import jax.numpy as jnp
from jax import lax
from jax.experimental import pallas as pl
from jax.experimental.pallas import tpu as pltpu
from jax.experimental.pallas import tpu_sc as plsc

_B, _S, _D = 4, 4096, 768
_N = _B * _S
_NW = 32
_P = _S // _NW
_C = 16
_SLABS = (16, 16, 16, 16, 16, 16, 16, 16)
_OFFS = tuple(sum(_SLABS[:i]) for i in range(len(_SLABS)))
_NSLAB = len(_SLABS)
_NVREG = _D // 16


def _emb_kernel(x_flat, tok_table, pe):
    mesh = plsc.VectorSubcoreMesh(core_axis_name="c", subcore_axis_name="s")

    @functools.partial(
        pl.kernel,
        out_type=jax.ShapeDtypeStruct((_N, _D), jnp.float32),
        mesh=mesh,
        scratch_types=[
            [[pltpu.VMEM((_C, _D), jnp.float32) for _ in range(_B)]
             for _ in range(2)],
            [pltpu.VMEM((_C, _D), jnp.float32) for _ in range(2)],
            pltpu.VMEM((_B * _P,), jnp.int32),
            [pltpu.SemaphoreType.DMA for _ in range(2)],
            [pltpu.SemaphoreType.DMA for _ in range(2)],
            [pltpu.SemaphoreType.DMA for _ in range(2)],
            pltpu.SemaphoreType.DMA,
        ],
    )
    def body(x_hbm, table_hbm, pe_hbm, out_hbm, sets, pe_v, idx_v,
             gsems, wsems, pe_sems, idx_sem):
        wid = lax.axis_index("s") * 2 + lax.axis_index("c")
        s0 = wid * _P

        def start_gathers(t):
            p0, sz = _OFFS[t], _SLABS[t]
            return [pltpu.async_copy(
                        table_hbm.at[idx_v.at[pl.ds(b * _P + p0, sz)]],
                        sets[t % 2][b].at[pl.ds(0, sz)], gsems[t % 2])
                    for b in range(_B)]

        def start_writes(t):
            p0, sz = _OFFS[t], _SLABS[t]
            return [pltpu.async_copy(
                        sets[t % 2][b].at[pl.ds(0, sz)],
                        out_hbm.at[pl.ds(b * _S + s0 + p0, sz)],
                        wsems[t % 2])
                    for b in range(_B)]

        def start_pe_load(t):
            p0, sz = _OFFS[t], _SLABS[t]
            return pltpu.async_copy(
                pe_hbm.at[pl.ds(s0 + p0, sz)],
                pe_v[t % 2].at[pl.ds(0, sz)], pe_sems[t % 2])

        idx_d = [pltpu.async_copy(x_hbm.at[pl.ds(b * _S + s0, _P)],
                                  idx_v.at[pl.ds(b * _P, _P)], idx_sem)
                 for b in range(_B)]
        pe_d = {0: start_pe_load(0), 1: start_pe_load(1)}
        for d in idx_d:
            d.wait()
        gd = {0: start_gathers(0)}
        wd = {}

        for t in range(_NSLAB):
            for d in gd.pop(t):
                d.wait()
            if t + 1 < _NSLAB:
                if t >= 1:
                    for d in wd.pop(t - 1):
                        d.wait()
                gd[t + 1] = start_gathers(t + 1)
            pe_d.pop(t).wait()
            pev = pe_v[t % 2]
            bufs = sets[t % 2]

            @pl.loop(0, _SLABS[t])
            def _(r):
                for j in range(_NVREG):
                    v = pev[r, pl.ds(j * 16, 16)]
                    for b in range(_B):
                        plsc.addupdate(bufs[b].at[r, pl.ds(j * 16, 16)], v)

            if t + 2 < _NSLAB:
                pe_d[t + 2] = start_pe_load(t + 2)
            wd[t] = start_writes(t)
        for ds_ in wd.values():
            for d in ds_:
                d.wait()

    return body(x_flat, tok_table, pe)


def kernel(x, tok_table, pe):
    out = _emb_kernel(x.reshape(_N), tok_table, pe)
    return out.reshape(_B, _S, _D)
